# baseline (speedup 1.0000x reference)
import functools

import jax
import jax.numpy as jnp
from jax import lax
from jax.experimental import pallas as pl
from jax.experimental.pallas import tpu as pltpu
from jax.experimental.pallas import tpu_sc as plsc

NC = 2
NS = 16
LANES = 16
NW = NC * NS
CH = 16

_GATHER_1D = lax.GatherDimensionNumbers(
    offset_dims=(), collapsed_slice_dims=(0,), start_index_map=(0,))


def _lane_perm(x, perm):
  return lax.gather(x, perm[:, None], _GATHER_1D, slice_sizes=(1,),
                    mode=lax.GatherScatterMode.PROMISE_IN_BOUNDS)


def _sc_embed_ln(ids, tts, word_table, pos_table, tok_table, gamma, beta,
                 *, seq_len):
  n_tok = ids.shape[0]
  dim = word_table.shape[1]
  n_batch = n_tok // seq_len
  per_w = n_tok // NW
  rows_w = seq_len // NW
  cpb = rows_w // CH
  n_chunks = per_w // CH
  nvec = dim // LANES
  inv_dim = 1.0 / dim

  mesh = plsc.VectorSubcoreMesh(
      core_axis_name="c", subcore_axis_name="s",
      num_cores=NC, num_subcores=NS)

  @functools.partial(
      pl.kernel,
      out_type=jax.ShapeDtypeStruct((n_tok, dim), jnp.float32),
      mesh=mesh,
      scratch_types=[
          pltpu.VMEM((per_w,), jnp.int32),
          pltpu.VMEM((per_w,), jnp.int32),
          pltpu.VMEM((CH, dim), jnp.float32),
          pltpu.VMEM((CH, dim), jnp.float32),
          pltpu.VMEM((CH, dim), jnp.float32),
          pltpu.VMEM((rows_w, dim), jnp.float32),
          pltpu.VMEM((dim,), jnp.float32),
          pltpu.VMEM((dim,), jnp.float32),
          pltpu.VMEM((dim,), jnp.float32),
          pltpu.VMEM((dim,), jnp.float32),
          pltpu.VMEM((2, CH * LANES), jnp.float32),
          pltpu.SemaphoreType.DMA,
          pltpu.SemaphoreType.DMA,
          pltpu.SemaphoreType.DMA,
          pltpu.SemaphoreType.DMA,
          pltpu.SemaphoreType.DMA,
          pltpu.SemaphoreType.DMA,
      ],
  )
  def body(ids_hbm, tts_hbm, word_hbm, pos_hbm, tok_hbm, gamma_hbm, beta_hbm,
           out_hbm, idxa, tta, r0, r1, r2, pos2_v, tokb_v, tokd_v,
           gamma_v, beta_v, stats_v, sg0, sg1, sg2, so0, so1, so2):
    wid = lax.axis_index("s") * NC + lax.axis_index("c")
    s_base = wid * rows_w

    def prologue_descs():
      descs = [
          pltpu.make_async_copy(gamma_hbm, gamma_v, sg0),
          pltpu.make_async_copy(beta_hbm, beta_v, sg0),
          pltpu.make_async_copy(tok_hbm.at[0], tokb_v, sg0),
          pltpu.make_async_copy(tok_hbm.at[1], tokd_v, sg0),
          pltpu.make_async_copy(pos_hbm.at[pl.ds(s_base, rows_w)], pos2_v,
                                sg0),
      ]
      for b in range(n_batch):
        src = pl.ds(b * seq_len + s_base, rows_w)
        dst = pl.ds(b * rows_w, rows_w)
        descs.append(pltpu.make_async_copy(ids_hbm.at[src], idxa.at[dst], sg0))
        descs.append(pltpu.make_async_copy(tts_hbm.at[src], tta.at[dst], sg0))
      return descs

    for d in prologue_descs():
      d.start()
    for d in prologue_descs():
      d.wait()
    for j in range(nvec):
      sl = pl.ds(j * LANES, LANES)
      tokd_v[sl] = tokd_v[sl] - tokb_v[sl]

    def pos_row(t, _):
      def lds(j):
        sl = pl.ds(j * LANES, LANES)
        return (pos2_v[t, sl], tokb_v[sl], sl)

      prev = lds(0)
      for j in range(1, nvec):
        cur = lds(j)
        pos2_v[t, prev[2]] = prev[0] + prev[1]
        prev = cur
      pos2_v[t, prev[2]] = prev[0] + prev[1]
      return 0

    lax.fori_loop(0, rows_w, pos_row, 0)

    slots = ((r0, sg0, so0), (r1, sg1, so1), (r2, sg2, so2))

    def tb_of(c):
      return (c // cpb) * seq_len + s_base + lax.rem(c, cpb) * CH

    def g_desc(c, rows, sg):
      return pltpu.make_async_copy(
          word_hbm.at[idxa.at[pl.ds(c * CH, CH)]], rows, sg)

    def out_desc(c, rows, so):
      return pltpu.make_async_copy(rows, out_hbm.at[pl.ds(tb_of(c), CH)], so)

    GRP = 4
    n_grp = nvec // GRP

    def compute(c, rows):
      ttv16 = tta[pl.ds(c * CH, CH)]
      p_base = lax.rem(c, cpb) * CH

      def token_body(t, _):
        ttf = _lane_perm(ttv16, jnp.full((LANES,), t, jnp.int32)).astype(
            jnp.float32)

        def load1(g):
          out = []
          for u in range(GRP):
            sl = pl.ds((g * GRP + u) * LANES, LANES)
            out.append((rows[t, sl], pos2_v[p_base + t, sl], tokd_v[sl], sl))
          return out

        accs = [jnp.zeros((LANES,), jnp.float32) for _ in range(GRP)]
        accq = [jnp.zeros((LANES,), jnp.float32) for _ in range(GRP)]

        def consume1(vals):
          for u, (w, p, td, sl) in enumerate(vals):
            x = (w + p) + ttf * td
            rows[t, sl] = x
            accs[u] = accs[u] + x
            accq[u] = accq[u] + x * x

        prev = load1(0)
        for g in range(1, n_grp):
          cur = load1(g)
          consume1(prev)
          prev = cur
        consume1(prev)

        acc_s = (accs[0] + accs[1]) + (accs[2] + accs[3])
        acc_q = (accq[0] + accq[1]) + (accq[2] + accq[3])
        lanes = lax.iota(jnp.int32, LANES)
        for sh in (8, 4, 2, 1):
          perm = lanes ^ sh
          acc_s = acc_s + _lane_perm(acc_s, perm)
          acc_q = acc_q + _lane_perm(acc_q, perm)
        muv = acc_s * inv_dim
        vv = acc_q * inv_dim - muv * muv + 1e-12
        seed = jnp.int32(0x5F3759DF) - (
            lax.bitcast_convert_type(vv, jnp.int32) >> 1)
        y = lax.bitcast_convert_type(seed, jnp.float32)
        for _ in range(2):
          y = y * (1.5 - 0.5 * vv * y * y)
        st = pl.ds(t * LANES, LANES)
        stats_v[0, st] = muv
        stats_v[1, st] = y
        return 0

      lax.fori_loop(0, CH, token_body, 0)

      mus = [stats_v[0, pl.ds(t * LANES, LANES)] for t in range(CH)]
      ys = [stats_v[1, pl.ds(t * LANES, LANES)] for t in range(CH)]

      def norm_body(j, _):
        for u in range(2):
          sl = pl.ds((2 * j + u) * LANES, LANES)
          gmm = gamma_v[sl]
          bta = beta_v[sl]
          for t in range(CH):
            x = rows[t, sl]
            rows[t, sl] = ((x - mus[t]) * ys[t]) * gmm + bta
        return 0

      lax.fori_loop(0, nvec // 2, norm_body, 0)

    g_desc(0, r0, sg0).start()

    def tri_body(k, _):
      for b in (0, 1, 2):
        rows, sg, so = slots[b]
        nrows, nsg, nso = slots[(b + 1) % 3]
        c = 3 * k + b
        g_desc(c, rows, sg).wait()

        @pl.when(c >= 2)
        def _():
          out_desc(c, nrows, nso).wait()

        g_desc(c + 1, nrows, nsg).start()
        compute(c, rows)
        out_desc(c, rows, so).start()
      return 0

    lax.fori_loop(0, (n_chunks - 1) // 3, tri_body, 0)
    c_last = n_chunks - 1
    rows, sg, so = slots[c_last % 3]
    g_desc(c_last, rows, sg).wait()
    compute(c_last, rows)
    out_desc(c_last, rows, so).start()
    for c in (n_chunks - 3, n_chunks - 2, n_chunks - 1):
      rows, sg, so = slots[c % 3]
      out_desc(c, rows, so).wait()

  return body(ids, tts, word_table, pos_table, tok_table, gamma, beta)


def kernel(input_ids, token_type_ids, word_table, pos_table, tok_table,
           gamma, beta):
  b, s = input_ids.shape
  dim = word_table.shape[1]
  ids = input_ids.reshape(b * s).astype(jnp.int32)
  tts = token_type_ids.reshape(b * s).astype(jnp.int32)
  out = _sc_embed_ln(ids, tts, word_table.astype(jnp.float32),
                     pos_table.astype(jnp.float32),
                     tok_table.astype(jnp.float32),
                     gamma.astype(jnp.float32), beta.astype(jnp.float32),
                     seq_len=s)
  return out.reshape(b, s, dim)


# speedup vs baseline: 1.5618x; 1.5618x over previous
import functools

import jax
import jax.numpy as jnp
from jax import lax
from jax.experimental import pallas as pl
from jax.experimental.pallas import tpu as pltpu
from jax.experimental.pallas import tpu_sc as plsc

NC = 2
NS = 16
LANES = 16
NW = NC * NS
CH = 16

_GATHER_1D = lax.GatherDimensionNumbers(
    offset_dims=(), collapsed_slice_dims=(0,), start_index_map=(0,))


def _lane_perm(x, perm):
  return lax.gather(x, perm[:, None], _GATHER_1D, slice_sizes=(1,),
                    mode=lax.GatherScatterMode.PROMISE_IN_BOUNDS)


def _sc_embed_ln(ids, tts, word_table, pos_table, tok_table, gamma, beta,
                 *, seq_len):
  n_tok = ids.shape[0]
  dim = word_table.shape[1]
  n_batch = n_tok // seq_len
  per_w = n_tok // NW
  rows_w = seq_len // NW
  cpb = rows_w // CH
  n_chunks = per_w // CH
  nvec = dim // LANES
  inv_dim = 1.0 / dim

  mesh = plsc.VectorSubcoreMesh(
      core_axis_name="c", subcore_axis_name="s",
      num_cores=NC, num_subcores=NS)

  @functools.partial(
      pl.kernel,
      out_type=jax.ShapeDtypeStruct((n_tok, dim), jnp.float32),
      mesh=mesh,
      scratch_types=[
          pltpu.VMEM((per_w,), jnp.int32),
          pltpu.VMEM((per_w,), jnp.int32),
          pltpu.VMEM((CH, dim), jnp.float32),
          pltpu.VMEM((CH, dim), jnp.float32),
          pltpu.VMEM((CH, dim), jnp.float32),
          pltpu.VMEM((rows_w, dim), jnp.float32),
          pltpu.VMEM((dim,), jnp.float32),
          pltpu.VMEM((dim,), jnp.float32),
          pltpu.VMEM((dim,), jnp.float32),
          pltpu.VMEM((dim,), jnp.float32),
          pltpu.VMEM((2, CH * LANES), jnp.float32),
          pltpu.SemaphoreType.DMA,
          pltpu.SemaphoreType.DMA,
          pltpu.SemaphoreType.DMA,
          pltpu.SemaphoreType.DMA,
          pltpu.SemaphoreType.DMA,
          pltpu.SemaphoreType.DMA,
      ],
  )
  def body(ids_hbm, tts_hbm, word_hbm, pos_hbm, tok_hbm, gamma_hbm, beta_hbm,
           out_hbm, idxa, tta, r0, r1, r2, pos2_v, tokb_v, tokd_v,
           gamma_v, beta_v, stats_v, sg0, sg1, sg2, so0, so1, so2):
    wid = lax.axis_index("s") * NC + lax.axis_index("c")
    s_base = wid * rows_w

    def prologue_descs():
      descs = [
          pltpu.make_async_copy(gamma_hbm, gamma_v, sg0),
          pltpu.make_async_copy(beta_hbm, beta_v, sg0),
          pltpu.make_async_copy(tok_hbm.at[0], tokb_v, sg0),
          pltpu.make_async_copy(tok_hbm.at[1], tokd_v, sg0),
          pltpu.make_async_copy(pos_hbm.at[pl.ds(s_base, rows_w)], pos2_v,
                                sg0),
      ]
      for b in range(n_batch):
        src = pl.ds(b * seq_len + s_base, rows_w)
        dst = pl.ds(b * rows_w, rows_w)
        descs.append(pltpu.make_async_copy(ids_hbm.at[src], idxa.at[dst], sg0))
        descs.append(pltpu.make_async_copy(tts_hbm.at[src], tta.at[dst], sg0))
      return descs

    for d in prologue_descs():
      d.start()
    for d in prologue_descs():
      d.wait()
    for j in range(nvec):
      sl = pl.ds(j * LANES, LANES)
      tokd_v[sl] = tokd_v[sl] - tokb_v[sl]

    def pos_row(t, _):
      def lds(j):
        sl = pl.ds(j * LANES, LANES)
        return (pos2_v[t, sl], tokb_v[sl], sl)

      prev = lds(0)
      for j in range(1, nvec):
        cur = lds(j)
        pos2_v[t, prev[2]] = prev[0] + prev[1]
        prev = cur
      pos2_v[t, prev[2]] = prev[0] + prev[1]
      return 0

    lax.fori_loop(0, rows_w, pos_row, 0)

    slots = ((r0, sg0, so0), (r1, sg1, so1), (r2, sg2, so2))

    def tb_of(c):
      return (c // cpb) * seq_len + s_base + lax.rem(c, cpb) * CH

    def g_desc(c, rows, sg):
      return pltpu.make_async_copy(
          word_hbm.at[idxa.at[pl.ds(c * CH, CH)]], rows, sg)

    def out_desc(c, rows, so):
      return pltpu.make_async_copy(rows, out_hbm.at[pl.ds(tb_of(c), CH)], so)

    GRP = 4
    n_grp = nvec // GRP

    def compute(c, rows):
      ttv16 = tta[pl.ds(c * CH, CH)]
      p_base = lax.rem(c, cpb) * CH

      def token_body(t, _):
        ttf = _lane_perm(ttv16, jnp.full((LANES,), t, jnp.int32)).astype(
            jnp.float32)

        def load1(g):
          out = []
          for u in range(GRP):
            sl = pl.ds((g * GRP + u) * LANES, LANES)
            out.append((rows[t, sl], pos2_v[p_base + t, sl], tokd_v[sl], sl))
          return out

        accs = [jnp.zeros((LANES,), jnp.float32) for _ in range(GRP)]
        accq = [jnp.zeros((LANES,), jnp.float32) for _ in range(GRP)]

        def consume1(vals):
          for u, (w, p, td, sl) in enumerate(vals):
            x = (w + p) + ttf * td
            rows[t, sl] = x
            accs[u] = accs[u] + x
            accq[u] = accq[u] + x * x

        prev = load1(0)
        for g in range(1, n_grp):
          cur = load1(g)
          consume1(prev)
          prev = cur
        consume1(prev)

        acc_s = (accs[0] + accs[1]) + (accs[2] + accs[3])
        acc_q = (accq[0] + accq[1]) + (accq[2] + accq[3])
        lanes = lax.iota(jnp.int32, LANES)
        for sh in (8, 4, 2, 1):
          perm = lanes ^ sh
          acc_s = acc_s + _lane_perm(acc_s, perm)
          acc_q = acc_q + _lane_perm(acc_q, perm)
        muv = acc_s * inv_dim
        vv = acc_q * inv_dim - muv * muv + 1e-12
        seed = jnp.int32(0x5F3759DF) - (
            lax.bitcast_convert_type(vv, jnp.int32) >> 1)
        y = lax.bitcast_convert_type(seed, jnp.float32)
        for _ in range(2):
          y = y * (1.5 - 0.5 * vv * y * y)
        st = pl.ds(t * LANES, LANES)
        stats_v[0, st] = muv
        stats_v[1, st] = y
        return 0

      lax.fori_loop(0, CH, token_body, 0)

      for h in range(2):
        t0 = h * (CH // 2)
        mus = [stats_v[0, pl.ds((t0 + t) * LANES, LANES)]
               for t in range(CH // 2)]
        ys = [stats_v[1, pl.ds((t0 + t) * LANES, LANES)]
              for t in range(CH // 2)]

        def norm_body(j, _):
          sl = pl.ds(j * LANES, LANES)
          gmm = gamma_v[sl]
          bta = beta_v[sl]
          for t in range(CH // 2):
            x = rows[t0 + t, sl]
            rows[t0 + t, sl] = ((x - mus[t]) * ys[t]) * gmm + bta
          return 0

        lax.fori_loop(0, nvec, norm_body, 0)

    g_desc(0, r0, sg0).start()

    def tri_body(k, _):
      for b in (0, 1, 2):
        rows, sg, so = slots[b]
        nrows, nsg, nso = slots[(b + 1) % 3]
        c = 3 * k + b
        g_desc(c, rows, sg).wait()

        @pl.when(c >= 2)
        def _():
          out_desc(c, nrows, nso).wait()

        g_desc(c + 1, nrows, nsg).start()
        compute(c, rows)
        out_desc(c, rows, so).start()
      return 0

    lax.fori_loop(0, (n_chunks - 1) // 3, tri_body, 0)
    c_last = n_chunks - 1
    rows, sg, so = slots[c_last % 3]
    g_desc(c_last, rows, sg).wait()
    compute(c_last, rows)
    out_desc(c_last, rows, so).start()
    for c in (n_chunks - 3, n_chunks - 2, n_chunks - 1):
      rows, sg, so = slots[c % 3]
      out_desc(c, rows, so).wait()

  return body(ids, tts, word_table, pos_table, tok_table, gamma, beta)


def kernel(input_ids, token_type_ids, word_table, pos_table, tok_table,
           gamma, beta):
  b, s = input_ids.shape
  dim = word_table.shape[1]
  ids = input_ids.reshape(b * s).astype(jnp.int32)
  tts = token_type_ids.reshape(b * s).astype(jnp.int32)
  out = _sc_embed_ln(ids, tts, word_table.astype(jnp.float32),
                     pos_table.astype(jnp.float32),
                     tok_table.astype(jnp.float32),
                     gamma.astype(jnp.float32), beta.astype(jnp.float32),
                     seq_len=s)
  return out.reshape(b, s, dim)


# speedup vs baseline: 1.7014x; 1.0894x over previous
import functools

import jax
import jax.numpy as jnp
from jax import lax
from jax.experimental import pallas as pl
from jax.experimental.pallas import tpu as pltpu
from jax.experimental.pallas import tpu_sc as plsc

NC = 2
NS = 16
LANES = 16
NW = NC * NS
CH = 16

_GATHER_1D = lax.GatherDimensionNumbers(
    offset_dims=(), collapsed_slice_dims=(0,), start_index_map=(0,))


def _lane_perm(x, perm):
  return lax.gather(x, perm[:, None], _GATHER_1D, slice_sizes=(1,),
                    mode=lax.GatherScatterMode.PROMISE_IN_BOUNDS)


def _sc_embed_ln(ids, tts, word_table, pos_table, tok_table, gamma, beta,
                 *, seq_len):
  n_tok = ids.shape[0]
  dim = word_table.shape[1]
  per_w = n_tok // NW
  n_chunks = per_w // CH
  nvec = dim // LANES
  inv_dim = 1.0 / dim

  mesh = plsc.VectorSubcoreMesh(
      core_axis_name="c", subcore_axis_name="s",
      num_cores=NC, num_subcores=NS)

  @functools.partial(
      pl.kernel,
      out_type=jax.ShapeDtypeStruct((n_tok, dim), jnp.float32),
      mesh=mesh,
      scratch_types=[
          pltpu.VMEM((per_w,), jnp.int32),
          pltpu.VMEM((per_w,), jnp.int32),
          pltpu.VMEM((CH, dim), jnp.float32),
          pltpu.VMEM((CH, dim), jnp.float32),
          pltpu.VMEM((CH, dim), jnp.float32),
          pltpu.VMEM((CH, dim), jnp.float32),
          pltpu.VMEM((CH, dim), jnp.float32),
          pltpu.VMEM((CH, dim), jnp.float32),
          pltpu.VMEM((dim,), jnp.float32),
          pltpu.VMEM((dim,), jnp.float32),
          pltpu.VMEM((dim,), jnp.float32),
          pltpu.VMEM((dim,), jnp.float32),
          pltpu.SemaphoreType.DMA,
          pltpu.SemaphoreType.DMA,
          pltpu.SemaphoreType.DMA,
          pltpu.SemaphoreType.DMA,
      ],
  )
  def body(ids_hbm, tts_hbm, word_hbm, pos_hbm, tok_hbm, gamma_hbm, beta_hbm,
           out_hbm, idxa, tta, r0, r1, p0, p1, o0, o1,
           tokb_v, tokd_v, gamma_v, beta_v, sg0, sg1, so0, so1):
    wid = lax.axis_index("s") * NC + lax.axis_index("c")
    base = wid * per_w

    def prologue_descs():
      return [
          pltpu.make_async_copy(gamma_hbm, gamma_v, sg0),
          pltpu.make_async_copy(beta_hbm, beta_v, sg0),
          pltpu.make_async_copy(tok_hbm.at[0], tokb_v, sg0),
          pltpu.make_async_copy(tok_hbm.at[1], tokd_v, sg0),
          pltpu.make_async_copy(ids_hbm.at[pl.ds(base, per_w)], idxa, sg0),
          pltpu.make_async_copy(tts_hbm.at[pl.ds(base, per_w)], tta, sg0),
      ]

    for d in prologue_descs():
      d.start()
    for d in prologue_descs():
      d.wait()
    for j in range(nvec):
      sl = pl.ds(j * LANES, LANES)
      tokd_v[sl] = tokd_v[sl] - tokb_v[sl]

    slots = ((r0, p0, o0, sg0, so0), (r1, p1, o1, sg1, so1))

    def g_descs(c, rows, pos, sg):
      tb = base + c * CH
      s_b = lax.rem(tb, seq_len)
      d_pos = pltpu.make_async_copy(pos_hbm.at[pl.ds(s_b, CH)], pos, sg)
      d_wrd = pltpu.make_async_copy(word_hbm.at[idxa.at[pl.ds(c * CH, CH)]],
                                    rows, sg)
      return d_pos, d_wrd

    def issue_g(c, rows, pos, sg):
      for d in g_descs(c, rows, pos, sg):
        d.start()

    def wait_g(c, rows, pos, sg):
      for d in g_descs(c, rows, pos, sg):
        d.wait()

    def out_desc(c, outb, so):
      tb = base + c * CH
      return pltpu.make_async_copy(outb, out_hbm.at[pl.ds(tb, CH)], so)

    GRP = 4
    n_grp = nvec // GRP

    def compute(c, rows, pos, outb):
      ttv16 = tta[pl.ds(c * CH, CH)]

      def token_body(t, _):
        ttf = _lane_perm(ttv16, jnp.full((LANES,), t, jnp.int32)).astype(
            jnp.float32)

        def load1(g):
          out = []
          for u in range(GRP):
            sl = pl.ds((g * GRP + u) * LANES, LANES)
            out.append((rows[t, sl], pos[t, sl], tokb_v[sl], tokd_v[sl], sl))
          return out

        accs = [jnp.zeros((LANES,), jnp.float32) for _ in range(GRP)]
        accq = [jnp.zeros((LANES,), jnp.float32) for _ in range(GRP)]

        def consume1(vals):
          for u, (w, p, tb, td, sl) in enumerate(vals):
            x = (w + p) + (tb + ttf * td)
            outb[t, sl] = x
            accs[u] = accs[u] + x
            accq[u] = accq[u] + x * x

        prev = load1(0)
        for g in range(1, n_grp):
          cur = load1(g)
          consume1(prev)
          prev = cur
        consume1(prev)

        acc_s = (accs[0] + accs[1]) + (accs[2] + accs[3])
        acc_q = (accq[0] + accq[1]) + (accq[2] + accq[3])
        lanes = lax.iota(jnp.int32, LANES)
        for sh in (8, 4, 2, 1):
          perm = lanes ^ sh
          acc_s = acc_s + _lane_perm(acc_s, perm)
          acc_q = acc_q + _lane_perm(acc_q, perm)
        muv = acc_s * inv_dim
        vv = acc_q * inv_dim - muv * muv + 1e-12
        seed = jnp.int32(0x5F3759DF) - (
            lax.bitcast_convert_type(vv, jnp.int32) >> 1)
        y = lax.bitcast_convert_type(seed, jnp.float32)
        for _ in range(2):
          y = y * (1.5 - 0.5 * vv * y * y)

        def load2(g):
          out = []
          for u in range(GRP):
            sl = pl.ds((g * GRP + u) * LANES, LANES)
            out.append((outb[t, sl], gamma_v[sl], beta_v[sl], sl))
          return out

        def consume2(vals):
          for x, gmm, bta, sl in vals:
            outb[t, sl] = ((x - muv) * y) * gmm + bta

        prev = load2(0)
        for g in range(1, n_grp):
          cur = load2(g)
          consume2(prev)
          prev = cur
        consume2(prev)
        return 0

      lax.fori_loop(0, CH, token_body, 0)

    issue_g(0, r0, p0, sg0)
    issue_g(1, r1, p1, sg1)

    def pair_body(k, _):
      for b in (0, 1):
        rows, pos, outb, sg, so = slots[b]
        c = 2 * k + b
        wait_g(c, rows, pos, sg)

        @pl.when(c >= 2)
        def _():
          out_desc(c, outb, so).wait()

        compute(c, rows, pos, outb)
        out_desc(c, outb, so).start()

        @pl.when(c + 2 < n_chunks)
        def _():
          issue_g(c + 2, rows, pos, sg)
      return 0

    lax.fori_loop(0, n_chunks // 2, pair_body, 0)
    out_desc(n_chunks - 2, o0, so0).wait()
    out_desc(n_chunks - 1, o1, so1).wait()

  return body(ids, tts, word_table, pos_table, tok_table, gamma, beta)


def kernel(input_ids, token_type_ids, word_table, pos_table, tok_table,
           gamma, beta):
  b, s = input_ids.shape
  dim = word_table.shape[1]
  ids = input_ids.reshape(b * s).astype(jnp.int32)
  tts = token_type_ids.reshape(b * s).astype(jnp.int32)
  out = _sc_embed_ln(ids, tts, word_table.astype(jnp.float32),
                     pos_table.astype(jnp.float32),
                     tok_table.astype(jnp.float32),
                     gamma.astype(jnp.float32), beta.astype(jnp.float32),
                     seq_len=s)
  return out.reshape(b, s, dim)


# speedup vs baseline: 1.7101x; 1.0051x over previous
import functools

import jax
import jax.numpy as jnp
from jax import lax
from jax.experimental import pallas as pl
from jax.experimental.pallas import tpu as pltpu
from jax.experimental.pallas import tpu_sc as plsc

NC = 2
NS = 16
LANES = 16
NW = NC * NS
CH = 16

_GATHER_1D = lax.GatherDimensionNumbers(
    offset_dims=(), collapsed_slice_dims=(0,), start_index_map=(0,))


def _lane_perm(x, perm):
  return lax.gather(x, perm[:, None], _GATHER_1D, slice_sizes=(1,),
                    mode=lax.GatherScatterMode.PROMISE_IN_BOUNDS)


def _sc_embed_ln(ids, tts, word_table, pos_table, tok_table, gamma, beta,
                 *, seq_len):
  n_tok = ids.shape[0]
  dim = word_table.shape[1]
  per_w = n_tok // NW
  n_chunks = per_w // CH
  nvec = dim // LANES
  inv_dim = 1.0 / dim

  mesh = plsc.VectorSubcoreMesh(
      core_axis_name="c", subcore_axis_name="s",
      num_cores=NC, num_subcores=NS)

  @functools.partial(
      pl.kernel,
      out_type=jax.ShapeDtypeStruct((n_tok, dim), jnp.float32),
      mesh=mesh,
      scratch_types=[
          pltpu.VMEM((per_w,), jnp.int32),
          pltpu.VMEM((per_w,), jnp.int32),
          pltpu.VMEM((CH, dim), jnp.float32),
          pltpu.VMEM((CH, dim), jnp.float32),
          pltpu.VMEM((CH, dim), jnp.float32),
          pltpu.VMEM((CH, dim), jnp.float32),
          pltpu.VMEM((CH, dim), jnp.float32),
          pltpu.VMEM((CH, dim), jnp.float32),
          pltpu.VMEM((dim,), jnp.float32),
          pltpu.VMEM((dim,), jnp.float32),
          pltpu.VMEM((dim,), jnp.float32),
          pltpu.VMEM((dim,), jnp.float32),
          pltpu.VMEM((2, CH * LANES), jnp.float32),
          pltpu.SemaphoreType.DMA,
          pltpu.SemaphoreType.DMA,
          pltpu.SemaphoreType.DMA,
          pltpu.SemaphoreType.DMA,
      ],
  )
  def body(ids_hbm, tts_hbm, word_hbm, pos_hbm, tok_hbm, gamma_hbm, beta_hbm,
           out_hbm, idxa, tta, r0, r1, p0, p1, o0, o1,
           tokb_v, tokd_v, gamma_v, beta_v, stats_v, sg0, sg1, so0, so1):
    wid = lax.axis_index("s") * NC + lax.axis_index("c")
    base = wid * per_w

    def prologue_descs():
      return [
          pltpu.make_async_copy(gamma_hbm, gamma_v, sg0),
          pltpu.make_async_copy(beta_hbm, beta_v, sg0),
          pltpu.make_async_copy(tok_hbm.at[0], tokb_v, sg0),
          pltpu.make_async_copy(tok_hbm.at[1], tokd_v, sg0),
          pltpu.make_async_copy(ids_hbm.at[pl.ds(base, per_w)], idxa, sg0),
          pltpu.make_async_copy(tts_hbm.at[pl.ds(base, per_w)], tta, sg0),
      ]

    for d in prologue_descs():
      d.start()
    for d in prologue_descs():
      d.wait()
    for j in range(nvec):
      sl = pl.ds(j * LANES, LANES)
      tokd_v[sl] = tokd_v[sl] - tokb_v[sl]

    slots = ((r0, p0, o0, sg0, so0), (r1, p1, o1, sg1, so1))

    def g_descs(c, rows, pos, sg):
      tb = base + c * CH
      s_b = lax.rem(tb, seq_len)
      d_pos = pltpu.make_async_copy(pos_hbm.at[pl.ds(s_b, CH)], pos, sg)
      d_wrd = pltpu.make_async_copy(word_hbm.at[idxa.at[pl.ds(c * CH, CH)]],
                                    rows, sg)
      return d_pos, d_wrd

    def issue_g(c, rows, pos, sg):
      for d in g_descs(c, rows, pos, sg):
        d.start()

    def wait_g(c, rows, pos, sg):
      for d in g_descs(c, rows, pos, sg):
        d.wait()

    def out_desc(c, outb, so):
      tb = base + c * CH
      return pltpu.make_async_copy(outb, out_hbm.at[pl.ds(tb, CH)], so)

    GRP = 4
    n_grp = nvec // GRP

    def compute(c, rows, pos, outb):
      ttv16 = tta[pl.ds(c * CH, CH)]

      def token_body(t, _):
        ttf = _lane_perm(ttv16, jnp.full((LANES,), t, jnp.int32)).astype(
            jnp.float32)

        def load1(g):
          out = []
          for u in range(GRP):
            sl = pl.ds((g * GRP + u) * LANES, LANES)
            out.append((rows[t, sl], pos[t, sl], tokb_v[sl], tokd_v[sl], sl))
          return out

        accs = [jnp.zeros((LANES,), jnp.float32) for _ in range(GRP)]
        accq = [jnp.zeros((LANES,), jnp.float32) for _ in range(GRP)]

        def consume1(vals):
          for u, (w, p, tb, td, sl) in enumerate(vals):
            x = (w + p) + (tb + ttf * td)
            outb[t, sl] = x
            accs[u] = accs[u] + x
            accq[u] = accq[u] + x * x

        prev = load1(0)
        for g in range(1, n_grp):
          cur = load1(g)
          consume1(prev)
          prev = cur
        consume1(prev)

        acc_s = (accs[0] + accs[1]) + (accs[2] + accs[3])
        acc_q = (accq[0] + accq[1]) + (accq[2] + accq[3])
        lanes = lax.iota(jnp.int32, LANES)
        for sh in (8, 4, 2, 1):
          perm = lanes ^ sh
          acc_s = acc_s + _lane_perm(acc_s, perm)
          acc_q = acc_q + _lane_perm(acc_q, perm)
        muv = acc_s * inv_dim
        vv = acc_q * inv_dim - muv * muv + 1e-12
        seed = jnp.int32(0x5F3759DF) - (
            lax.bitcast_convert_type(vv, jnp.int32) >> 1)
        y = lax.bitcast_convert_type(seed, jnp.float32)
        for _ in range(2):
          y = y * (1.5 - 0.5 * vv * y * y)
        st = pl.ds(t * LANES, LANES)
        stats_v[0, st] = muv
        stats_v[1, st] = y
        return 0

      lax.fori_loop(0, CH, token_body, 0)

      QT = 4

      def quarter_body(q, _):
        t0 = q * QT
        mus = [stats_v[0, pl.ds((t0 + i) * LANES, LANES)] for i in range(QT)]
        ys = [stats_v[1, pl.ds((t0 + i) * LANES, LANES)] for i in range(QT)]

        def load2(j):
          sl = pl.ds(j * LANES, LANES)
          return (gamma_v[sl], beta_v[sl],
                  [outb[t0 + i, sl] for i in range(QT)], sl)

        def consume2(vals):
          gmm, bta, xs, sl = vals
          for i in range(QT):
            outb[t0 + i, sl] = ((xs[i] - mus[i]) * ys[i]) * gmm + bta

        prev = load2(0)
        for j in range(1, nvec):
          cur = load2(j)
          consume2(prev)
          prev = cur
        consume2(prev)
        return 0

      lax.fori_loop(0, CH // QT, quarter_body, 0)

    issue_g(0, r0, p0, sg0)
    issue_g(1, r1, p1, sg1)

    def pair_body(k, _):
      for b in (0, 1):
        rows, pos, outb, sg, so = slots[b]
        c = 2 * k + b
        wait_g(c, rows, pos, sg)

        @pl.when(c >= 2)
        def _():
          out_desc(c, outb, so).wait()

        compute(c, rows, pos, outb)
        out_desc(c, outb, so).start()

        @pl.when(c + 2 < n_chunks)
        def _():
          issue_g(c + 2, rows, pos, sg)
      return 0

    lax.fori_loop(0, n_chunks // 2, pair_body, 0)
    out_desc(n_chunks - 2, o0, so0).wait()
    out_desc(n_chunks - 1, o1, so1).wait()

  return body(ids, tts, word_table, pos_table, tok_table, gamma, beta)


def kernel(input_ids, token_type_ids, word_table, pos_table, tok_table,
           gamma, beta):
  b, s = input_ids.shape
  dim = word_table.shape[1]
  ids = input_ids.reshape(b * s).astype(jnp.int32)
  tts = token_type_ids.reshape(b * s).astype(jnp.int32)
  out = _sc_embed_ln(ids, tts, word_table.astype(jnp.float32),
                     pos_table.astype(jnp.float32),
                     tok_table.astype(jnp.float32),
                     gamma.astype(jnp.float32), beta.astype(jnp.float32),
                     seq_len=s)
  return out.reshape(b, s, dim)
